# baseline (speedup 1.0000x reference)
import functools

import jax
import jax.numpy as jnp
from jax import lax
from jax.experimental import pallas as pl
from jax.experimental.pallas import tpu as pltpu
from jax.experimental.pallas import tpu_sc as plsc

N_CORES = 2
N_SUBCORES = 16
N_TILES = N_CORES * N_SUBCORES
LANES = 16

_MESH = plsc.VectorSubcoreMesh(core_axis_name="c", subcore_axis_name="s")


def _deg_body(dst_hbm, out_hbm, idx_v, deg_v, sem):
    E = dst_hbm.shape[0]
    n = out_hbm.shape[1]
    per_tile = E // N_TILES
    cid = lax.axis_index("c")
    sid = lax.axis_index("s")
    wid = cid * N_SUBCORES + sid

    pltpu.async_copy(dst_hbm.at[pl.ds(wid * per_tile, per_tile)], idx_v, sem).wait()

    zeros16 = jnp.zeros((LANES,), jnp.float32)
    ones16 = jnp.ones((LANES,), jnp.float32)

    def zero_body(i, _):
        deg_v[pl.ds(i * LANES, LANES)] = zeros16
        return 0

    lax.fori_loop(0, n // LANES, zero_body, 0)

    def acc_body(i, _):
        idx = idx_v[pl.ds(i * LANES, LANES)]
        plsc.addupdate_scatter(deg_v, [idx], ones16)
        return 0

    lax.fori_loop(0, per_tile // LANES, acc_body, 0)

    pltpu.sync_copy(deg_v, out_hbm.at[wid])


def _deg_partials(dst, n_nodes):
    E = dst.shape[0]
    kern = pl.kernel(
        _deg_body,
        out_type=jax.ShapeDtypeStruct((N_TILES, n_nodes), jnp.float32),
        mesh=_MESH,
        compiler_params=pltpu.CompilerParams(needs_layout_passes=False),
        scratch_types=[
            pltpu.VMEM((E // N_TILES,), jnp.int32),
            pltpu.VMEM((n_nodes,), jnp.float32),
            pltpu.SemaphoreType.DMA,
        ],
    )
    return kern(dst)


def _agg_body(g_hbm, src_hbm, dst_hbm, part_hbm,
              rows_a, rows_b, idx_s, idx_d, acc,
              sem_a, sem_b):
    n, ch = g_hbm.shape
    chunk = 128
    grp = idx_s.shape[0]
    n_chunks = src_hbm.shape[0]
    per_tile = n_chunks // N_TILES
    n_grp = per_tile // grp
    stripe = (n // N_SUBCORES) // 8 * 8
    leftover = n - stripe * N_SUBCORES

    cid = lax.axis_index("c")
    sid = lax.axis_index("s")
    wid = cid * N_SUBCORES + sid
    base_row = wid * per_tile

    zeros16 = jnp.zeros((LANES,), jnp.float32)

    def zrow(i, _):
        for j in range(ch // LANES):
            rows_a[i, pl.ds(j * LANES, LANES)] = zeros16
        return 0

    lax.fori_loop(0, chunk, zrow, 0)

    row0 = sid * stripe
    n_zfull = stripe // chunk
    zrem = stripe % chunk
    for k in range(n_zfull):
        pltpu.sync_copy(rows_a, acc.at[pl.ds(row0 + k * chunk, chunk)])
    if zrem:
        pltpu.sync_copy(rows_a.at[pl.ds(0, zrem)],
                        acc.at[pl.ds(row0 + n_zfull * chunk, zrem)])
    if leftover:
        @pl.when(sid == N_SUBCORES - 1)
        def _():
            pltpu.sync_copy(rows_a.at[pl.ds(0, leftover)],
                            acc.at[pl.ds(stripe * N_SUBCORES, leftover)])

    plsc.subcore_barrier()

    bufs = (rows_a, rows_b)
    gsems = (sem_a, sem_b)

    def group(gi, _):
        row = base_row + gi * grp
        pltpu.sync_copy(src_hbm.at[pl.ds(row, grp), :], idx_s)
        pltpu.sync_copy(dst_hbm.at[pl.ds(row, grp), :], idx_d)
        g_pend = pltpu.async_copy(
            g_hbm.at[idx_s.at[0]], bufs[0], gsems[0])
        for k in range(grp):
            if k + 1 < grp:
                g_next = pltpu.async_copy(
                    g_hbm.at[idx_s.at[k + 1]],
                    bufs[(k + 1) % 2], gsems[(k + 1) % 2])
            g_pend.wait()
            pltpu.sync_copy(bufs[k % 2], acc.at[idx_d.at[k]], add=True)
            if k + 1 < grp:
                g_pend = g_next
        return 0

    lax.fori_loop(0, n_grp, group, 0)

    plsc.subcore_barrier()

    pltpu.sync_copy(acc.at[pl.ds(row0, stripe)],
                    part_hbm.at[cid, pl.ds(row0, stripe), :])
    if leftover:
        @pl.when(sid == N_SUBCORES - 1)
        def _():
            pltpu.sync_copy(
                acc.at[pl.ds(stripe * N_SUBCORES, leftover)],
                part_hbm.at[cid, pl.ds(stripe * N_SUBCORES, leftover), :])


def _aggregate(g, src2, dst2):
    n, ch = g.shape
    chunk = 128
    grp = 8
    kern = pl.kernel(
        _agg_body,
        out_type=jax.ShapeDtypeStruct((N_CORES, n, ch), jnp.float32),
        mesh=_MESH,
        scratch_types=[
            pltpu.VMEM((chunk, ch), jnp.float32),
            pltpu.VMEM((chunk, ch), jnp.float32),
            pltpu.VMEM((grp, chunk), jnp.int32),
            pltpu.VMEM((grp, chunk), jnp.int32),
            pltpu.VMEM_SHARED((n + 8, ch), jnp.float32),
            pltpu.SemaphoreType.DMA,
            pltpu.SemaphoreType.DMA,
        ],
    )
    return kern(g, src2, dst2)


_BLK = 1000


def _first_body(degp_ref, x_ref, w_ref, g_ref, dinv_ref):
    deg = jnp.sum(degp_ref[...], axis=1) + 1.0
    dinv = lax.rsqrt(deg)
    h = jnp.dot(x_ref[...], w_ref[...], preferred_element_type=jnp.float32)
    g_ref[...] = h * dinv[:, None]
    dinv_ref[...] = dinv[:, None]


def _first_stage(degp, x, w):
    n, ch_in = x.shape
    ch_out = w.shape[1]
    grid = (n // _BLK,)
    return pl.pallas_call(
        _first_body,
        grid=grid,
        in_specs=[
            pl.BlockSpec((_BLK, N_TILES), lambda i: (i, 0)),
            pl.BlockSpec((_BLK, ch_in), lambda i: (i, 0)),
            pl.BlockSpec((ch_in, ch_out), lambda i: (0, 0)),
        ],
        out_specs=[
            pl.BlockSpec((_BLK, ch_out), lambda i: (i, 0)),
            pl.BlockSpec((_BLK, 1), lambda i: (i, 0)),
        ],
        out_shape=[
            jax.ShapeDtypeStruct((n, ch_out), jnp.float32),
            jax.ShapeDtypeStruct((n, 1), jnp.float32),
        ],
    )(degp, x, w)


def _mid_body(part_ref, g_ref, dinv_ref, b_ref, w_ref, gn_ref):
    agg = part_ref[0] + part_ref[1] + g_ref[...]
    h = agg * dinv_ref[...] + b_ref[...]
    h = jnp.maximum(h, 0.0)
    hw = jnp.dot(h, w_ref[...], preferred_element_type=jnp.float32)
    gn_ref[...] = hw * dinv_ref[...]


def _mid_stage(part, g, dinv, b, w):
    n, ch = g.shape
    ch_out = w.shape[1]
    grid = (n // _BLK,)
    return pl.pallas_call(
        _mid_body,
        grid=grid,
        in_specs=[
            pl.BlockSpec((N_CORES, _BLK, ch), lambda i: (0, i, 0)),
            pl.BlockSpec((_BLK, ch), lambda i: (i, 0)),
            pl.BlockSpec((_BLK, 1), lambda i: (i, 0)),
            pl.BlockSpec((1, ch), lambda i: (0, 0)),
            pl.BlockSpec((ch, ch_out), lambda i: (0, 0)),
        ],
        out_specs=pl.BlockSpec((_BLK, ch_out), lambda i: (i, 0)),
        out_shape=jax.ShapeDtypeStruct((n, ch_out), jnp.float32),
    )(part, g, dinv, b.reshape(1, ch), w)


def _last_body(part_ref, g_ref, dinv_ref, b_ref, out_ref):
    agg = part_ref[0] + part_ref[1] + g_ref[...]
    out_ref[...] = agg * dinv_ref[...] + b_ref[...]


def _last_stage(part, g, dinv, b):
    n, ch = g.shape
    grid = (n // _BLK,)
    return pl.pallas_call(
        _last_body,
        grid=grid,
        in_specs=[
            pl.BlockSpec((N_CORES, _BLK, ch), lambda i: (0, i, 0)),
            pl.BlockSpec((_BLK, ch), lambda i: (i, 0)),
            pl.BlockSpec((_BLK, 1), lambda i: (i, 0)),
            pl.BlockSpec((1, ch), lambda i: (0, 0)),
        ],
        out_specs=pl.BlockSpec((_BLK, ch), lambda i: (i, 0)),
        out_shape=jax.ShapeDtypeStruct((n, ch), jnp.float32),
    )(part, g, dinv, b.reshape(1, ch))


@jax.jit
def _run(x, edge_index, W1, b1, W2, b2, W_mu, b_mu, W_logstd, b_logstd):
    n = x.shape[0]
    src = edge_index[0]
    dst = edge_index[1]
    E = src.shape[0]

    tile_quota = N_TILES * 128 * 8
    E2 = ((E + tile_quota - 1) // tile_quota) * tile_quota
    pad = E2 - E
    src2 = jnp.concatenate(
        [src, jnp.zeros((pad,), jnp.int32)]).reshape(E2 // 128, 128)
    dst2 = jnp.concatenate(
        [dst, jnp.full((pad,), n, jnp.int32)]).reshape(E2 // 128, 128)

    degp = _deg_partials(dst, n)

    g1, dinv = _first_stage(degp.T, x, W1)
    part1 = _aggregate(g1, src2, dst2)

    g2 = _mid_stage(part1, g1, dinv, b1, W2)
    part2 = _aggregate(g2, src2, dst2)

    w_cat = jnp.concatenate([W_mu, W_logstd], axis=1)
    b_cat = jnp.concatenate([b_mu, b_logstd], axis=0)
    g3 = _mid_stage(part2, g2, dinv, b2, w_cat)
    part3 = _aggregate(g3, src2, dst2)

    out = _last_stage(part3, g3, dinv, b_cat)
    ch = W_mu.shape[1]
    return out[:, :ch], out[:, ch:]


def kernel(x, edge_index, W1, b1, W2, b2, W_mu, b_mu, W_logstd, b_logstd):
    return _run(x, edge_index, W1, b1, W2, b2, W_mu, b_mu, W_logstd, b_logstd)


# speedup vs baseline: 3.5234x; 3.5234x over previous
import functools

import jax
import jax.numpy as jnp
from jax import lax
from jax.experimental import pallas as pl
from jax.experimental.pallas import tpu as pltpu
from jax.experimental.pallas import tpu_sc as plsc

N_CORES = 2
N_SUBCORES = 16
N_TILES = N_CORES * N_SUBCORES
LANES = 16

_MESH = plsc.VectorSubcoreMesh(core_axis_name="c", subcore_axis_name="s")


def _deg_body(dst_hbm, out_hbm, idx_v, deg_v, sem):
    E = dst_hbm.shape[0]
    n = out_hbm.shape[1]
    per_tile = E // N_TILES
    cid = lax.axis_index("c")
    sid = lax.axis_index("s")
    wid = cid * N_SUBCORES + sid

    pltpu.async_copy(dst_hbm.at[pl.ds(wid * per_tile, per_tile)], idx_v, sem).wait()

    zeros16 = jnp.zeros((LANES,), jnp.float32)
    ones16 = jnp.ones((LANES,), jnp.float32)

    def zero_body(i, _):
        deg_v[pl.ds(i * LANES, LANES)] = zeros16
        return 0

    lax.fori_loop(0, n // LANES, zero_body, 0)

    def acc_body(i, _):
        idx = idx_v[pl.ds(i * LANES, LANES)]
        plsc.addupdate_scatter(deg_v, [idx], ones16)
        return 0

    lax.fori_loop(0, per_tile // LANES, acc_body, 0)

    pltpu.sync_copy(deg_v, out_hbm.at[wid])


def _deg_partials(dst, n_nodes):
    E = dst.shape[0]
    kern = pl.kernel(
        _deg_body,
        out_type=jax.ShapeDtypeStruct((N_TILES, n_nodes), jnp.float32),
        mesh=_MESH,
        compiler_params=pltpu.CompilerParams(needs_layout_passes=False),
        scratch_types=[
            pltpu.VMEM((E // N_TILES,), jnp.int32),
            pltpu.VMEM((n_nodes,), jnp.float32),
            pltpu.SemaphoreType.DMA,
        ],
    )
    return kern(dst)


def _agg_body(g_hbm, src_hbm, dst_hbm, part_hbm,
              rows_a, rows_b, idx_s, idx_da, idx_db,
              idx_st, idx_dt, acc,
              sem_a, sem_b, semd_a, semd_b):
    E = src_hbm.shape[0]
    n, ch = g_hbm.shape
    chunk = 128
    n_chunks = E // chunk
    n_full = n_chunks // N_TILES
    n_extra = n_chunks % N_TILES
    stripe = (n // N_SUBCORES) // 8 * 8
    leftover = n - stripe * N_SUBCORES

    cid = lax.axis_index("c")
    sid = lax.axis_index("s")
    wid = cid * N_SUBCORES + sid
    base = wid * n_full * chunk

    zeros16 = jnp.zeros((LANES,), jnp.float32)

    def zrow(i, _):
        for j in range(ch // LANES):
            rows_a[i, pl.ds(j * LANES, LANES)] = zeros16
        return 0

    lax.fori_loop(0, chunk, zrow, 0)

    row0 = sid * stripe
    n_zfull = stripe // chunk
    zrem = stripe % chunk
    for k in range(n_zfull):
        pltpu.sync_copy(rows_a, acc.at[pl.ds(row0 + k * chunk, chunk)])
    if zrem:
        pltpu.sync_copy(rows_a.at[pl.ds(0, zrem)],
                        acc.at[pl.ds(row0 + n_zfull * chunk, zrem)])
    if leftover:
        @pl.when(sid == N_SUBCORES - 1)
        def _():
            pltpu.sync_copy(rows_a.at[pl.ds(0, leftover)],
                            acc.at[pl.ds(stripe * N_SUBCORES, leftover)])

    plsc.subcore_barrier()

    grp = 13
    n_grp = n_full // grp
    assert n_grp * grp == n_full

    bufs = (rows_a, rows_b)
    didx = (idx_da, idx_db)
    gsems = (sem_a, sem_b)
    dsems = (semd_a, semd_b)

    def group(gi, _):
        off = base + gi * grp * chunk
        pltpu.sync_copy(src_hbm.at[pl.ds(off, grp * chunk)], idx_s)
        g_pend = pltpu.async_copy(
            g_hbm.at[idx_s.at[pl.ds(0, chunk)]], bufs[0], gsems[0])
        d_pend = pltpu.async_copy(
            dst_hbm.at[pl.ds(off, chunk)], didx[0], dsems[0])
        for k in range(grp):
            if k + 1 < grp:
                g_next = pltpu.async_copy(
                    g_hbm.at[idx_s.at[pl.ds((k + 1) * chunk, chunk)]],
                    bufs[(k + 1) % 2], gsems[(k + 1) % 2])
                d_next = pltpu.async_copy(
                    dst_hbm.at[pl.ds(off + (k + 1) * chunk, chunk)],
                    didx[(k + 1) % 2], dsems[(k + 1) % 2])
            g_pend.wait()
            d_pend.wait()
            pltpu.sync_copy(bufs[k % 2], acc.at[didx[k % 2]], add=True)
            if k + 1 < grp:
                g_pend = g_next
                d_pend = d_next
        return 0

    lax.fori_loop(0, n_grp, group, 0)

    if n_extra:
        @pl.when(wid < n_extra)
        def _():
            off = (N_TILES * n_full + wid) * chunk
            pltpu.sync_copy(src_hbm.at[pl.ds(off, chunk)], idx_st)
            pltpu.sync_copy(dst_hbm.at[pl.ds(off, chunk)], idx_dt)
            pltpu.async_copy(g_hbm.at[idx_st], rows_a, sem_a).wait()
            pltpu.sync_copy(rows_a, acc.at[idx_dt], add=True)

    plsc.subcore_barrier()

    pltpu.sync_copy(acc.at[pl.ds(row0, stripe)],
                    part_hbm.at[cid, pl.ds(row0, stripe), :])
    if leftover:
        @pl.when(sid == N_SUBCORES - 1)
        def _():
            pltpu.sync_copy(
                acc.at[pl.ds(stripe * N_SUBCORES, leftover)],
                part_hbm.at[cid, pl.ds(stripe * N_SUBCORES, leftover), :])


def _aggregate(g, src, dst):
    n, ch = g.shape
    chunk = 128
    grp = 13
    kern = pl.kernel(
        _agg_body,
        out_type=jax.ShapeDtypeStruct((N_CORES, n, ch), jnp.float32),
        mesh=_MESH,
        scratch_types=[
            pltpu.VMEM((chunk, ch), jnp.float32),
            pltpu.VMEM((chunk, ch), jnp.float32),
            pltpu.VMEM((grp * chunk,), jnp.int32),
            pltpu.VMEM((chunk,), jnp.int32),
            pltpu.VMEM((chunk,), jnp.int32),
            pltpu.VMEM((chunk,), jnp.int32),
            pltpu.VMEM((chunk,), jnp.int32),
            pltpu.VMEM_SHARED((n, ch), jnp.float32),
            pltpu.SemaphoreType.DMA,
            pltpu.SemaphoreType.DMA,
            pltpu.SemaphoreType.DMA,
            pltpu.SemaphoreType.DMA,
        ],
    )
    return kern(g, src, dst)


_BLK = 1000


def _first_body(degp_ref, x_ref, w_ref, g_ref, dinv_ref):
    deg = jnp.sum(degp_ref[...], axis=1) + 1.0
    dinv = lax.rsqrt(deg)
    h = jnp.dot(x_ref[...], w_ref[...], preferred_element_type=jnp.float32)
    g_ref[...] = h * dinv[:, None]
    dinv_ref[...] = dinv[:, None]


def _first_stage(degp, x, w):
    n, ch_in = x.shape
    ch_out = w.shape[1]
    grid = (n // _BLK,)
    return pl.pallas_call(
        _first_body,
        grid=grid,
        in_specs=[
            pl.BlockSpec((_BLK, N_TILES), lambda i: (i, 0)),
            pl.BlockSpec((_BLK, ch_in), lambda i: (i, 0)),
            pl.BlockSpec((ch_in, ch_out), lambda i: (0, 0)),
        ],
        out_specs=[
            pl.BlockSpec((_BLK, ch_out), lambda i: (i, 0)),
            pl.BlockSpec((_BLK, 1), lambda i: (i, 0)),
        ],
        out_shape=[
            jax.ShapeDtypeStruct((n, ch_out), jnp.float32),
            jax.ShapeDtypeStruct((n, 1), jnp.float32),
        ],
    )(degp, x, w)


def _mid_body(part_ref, g_ref, dinv_ref, b_ref, w_ref, gn_ref):
    agg = part_ref[0] + part_ref[1] + g_ref[...]
    h = agg * dinv_ref[...] + b_ref[...]
    h = jnp.maximum(h, 0.0)
    hw = jnp.dot(h, w_ref[...], preferred_element_type=jnp.float32)
    gn_ref[...] = hw * dinv_ref[...]


def _mid_stage(part, g, dinv, b, w):
    n, ch = g.shape
    ch_out = w.shape[1]
    grid = (n // _BLK,)
    return pl.pallas_call(
        _mid_body,
        grid=grid,
        in_specs=[
            pl.BlockSpec((N_CORES, _BLK, ch), lambda i: (0, i, 0)),
            pl.BlockSpec((_BLK, ch), lambda i: (i, 0)),
            pl.BlockSpec((_BLK, 1), lambda i: (i, 0)),
            pl.BlockSpec((1, ch), lambda i: (0, 0)),
            pl.BlockSpec((ch, ch_out), lambda i: (0, 0)),
        ],
        out_specs=pl.BlockSpec((_BLK, ch_out), lambda i: (i, 0)),
        out_shape=jax.ShapeDtypeStruct((n, ch_out), jnp.float32),
    )(part, g, dinv, b.reshape(1, ch), w)


def _last_body(part_ref, g_ref, dinv_ref, b_ref, out_ref):
    agg = part_ref[0] + part_ref[1] + g_ref[...]
    out_ref[...] = agg * dinv_ref[...] + b_ref[...]


def _last_stage(part, g, dinv, b):
    n, ch = g.shape
    grid = (n // _BLK,)
    return pl.pallas_call(
        _last_body,
        grid=grid,
        in_specs=[
            pl.BlockSpec((N_CORES, _BLK, ch), lambda i: (0, i, 0)),
            pl.BlockSpec((_BLK, ch), lambda i: (i, 0)),
            pl.BlockSpec((_BLK, 1), lambda i: (i, 0)),
            pl.BlockSpec((1, ch), lambda i: (0, 0)),
        ],
        out_specs=pl.BlockSpec((_BLK, ch), lambda i: (i, 0)),
        out_shape=jax.ShapeDtypeStruct((n, ch), jnp.float32),
    )(part, g, dinv, b.reshape(1, ch))


@jax.jit
def _run(x, edge_index, W1, b1, W2, b2, W_mu, b_mu, W_logstd, b_logstd):
    n = x.shape[0]
    src = edge_index[0]
    dst = edge_index[1]

    degp = _deg_partials(dst, n)

    g1, dinv = _first_stage(degp.T, x, W1)
    part1 = _aggregate(g1, src, dst)

    g2 = _mid_stage(part1, g1, dinv, b1, W2)
    part2 = _aggregate(g2, src, dst)

    w_cat = jnp.concatenate([W_mu, W_logstd], axis=1)
    b_cat = jnp.concatenate([b_mu, b_logstd], axis=0)
    g3 = _mid_stage(part2, g2, dinv, b2, w_cat)
    part3 = _aggregate(g3, src, dst)

    out = _last_stage(part3, g3, dinv, b_cat)
    ch = W_mu.shape[1]
    return out[:, :ch], out[:, ch:]


def kernel(x, edge_index, W1, b1, W2, b2, W_mu, b_mu, W_logstd, b_logstd):
    return _run(x, edge_index, W1, b1, W2, b2, W_mu, b_mu, W_logstd, b_logstd)


# speedup vs baseline: 3.8514x; 1.0931x over previous
import functools

import jax
import jax.numpy as jnp
from jax import lax
from jax.experimental import pallas as pl
from jax.experimental.pallas import tpu as pltpu
from jax.experimental.pallas import tpu_sc as plsc

N_CORES = 2
N_SUBCORES = 16
N_TILES = N_CORES * N_SUBCORES
LANES = 16

_MESH = plsc.VectorSubcoreMesh(core_axis_name="c", subcore_axis_name="s")


def _deg_body(dst_hbm, out_hbm, idx_v, deg_v, sem):
    E = dst_hbm.shape[0]
    n = out_hbm.shape[1]
    per_tile = E // N_TILES
    cid = lax.axis_index("c")
    sid = lax.axis_index("s")
    wid = cid * N_SUBCORES + sid

    pltpu.async_copy(dst_hbm.at[pl.ds(wid * per_tile, per_tile)], idx_v, sem).wait()

    zeros16 = jnp.zeros((LANES,), jnp.float32)
    ones16 = jnp.ones((LANES,), jnp.float32)

    def zero_body(i, _):
        deg_v[pl.ds(i * LANES, LANES)] = zeros16
        return 0

    lax.fori_loop(0, n // LANES, zero_body, 0)

    def acc_body(i, _):
        idx = idx_v[pl.ds(i * LANES, LANES)]
        plsc.addupdate_scatter(deg_v, [idx], ones16)
        return 0

    lax.fori_loop(0, per_tile // LANES, acc_body, 0)

    pltpu.sync_copy(deg_v, out_hbm.at[wid])


def _deg_partials(dst, n_nodes):
    E = dst.shape[0]
    kern = pl.kernel(
        _deg_body,
        out_type=jax.ShapeDtypeStruct((N_TILES, n_nodes), jnp.float32),
        mesh=_MESH,
        compiler_params=pltpu.CompilerParams(needs_layout_passes=False),
        scratch_types=[
            pltpu.VMEM((E // N_TILES,), jnp.int32),
            pltpu.VMEM((n_nodes,), jnp.float32),
            pltpu.SemaphoreType.DMA,
        ],
    )
    return kern(dst)


def _agg_body(g_hbm, src_hbm, dst_hbm, part_hbm,
              rows_a, rows_b, idx_s, idx_da, idx_db,
              idx_st, idx_dt, acc,
              sem_i, sem_a, sem_b, semd_a, semd_b):
    E = src_hbm.shape[0]
    n, ch = g_hbm.shape
    chunk = 128
    n_chunks = E // chunk
    n_full = n_chunks // N_TILES
    n_extra = n_chunks % N_TILES
    stripe = (n // N_SUBCORES) // 8 * 8
    leftover = n - stripe * N_SUBCORES

    cid = lax.axis_index("c")
    sid = lax.axis_index("s")
    wid = cid * N_SUBCORES + sid
    base = wid * n_full * chunk

    i_pend = pltpu.async_copy(
        src_hbm.at[pl.ds(base, n_full * chunk)], idx_s, sem_i)

    zeros16 = jnp.zeros((LANES,), jnp.float32)

    def zrow(i, _):
        for j in range(ch // LANES):
            rows_a[i, pl.ds(j * LANES, LANES)] = zeros16
        return 0

    lax.fori_loop(0, chunk, zrow, 0)

    row0 = sid * stripe
    n_zfull = stripe // chunk
    zrem = stripe % chunk
    for k in range(n_zfull):
        pltpu.sync_copy(rows_a, acc.at[pl.ds(row0 + k * chunk, chunk)])
    if zrem:
        pltpu.sync_copy(rows_a.at[pl.ds(0, zrem)],
                        acc.at[pl.ds(row0 + n_zfull * chunk, zrem)])
    if leftover:
        @pl.when(sid == N_SUBCORES - 1)
        def _():
            pltpu.sync_copy(rows_a.at[pl.ds(0, leftover)],
                            acc.at[pl.ds(stripe * N_SUBCORES, leftover)])

    i_pend.wait()
    pltpu.async_copy(g_hbm.at[idx_s.at[pl.ds(0, chunk)]], rows_a, sem_a)
    pltpu.async_copy(dst_hbm.at[pl.ds(base, chunk)], idx_da, semd_a)

    plsc.subcore_barrier()

    nb = n_full // 2
    assert nb * 2 == n_full

    def wait_a():
        pltpu.make_async_copy(
            g_hbm.at[idx_s.at[pl.ds(0, chunk)]], rows_a, sem_a).wait()
        pltpu.make_async_copy(
            dst_hbm.at[pl.ds(0, chunk)], idx_da, semd_a).wait()

    def body(i, _):
        off = base + 2 * i * chunk
        g_odd = pltpu.async_copy(
            g_hbm.at[idx_s.at[pl.ds((2 * i + 1) * chunk, chunk)]],
            rows_b, sem_b)
        d_odd = pltpu.async_copy(
            dst_hbm.at[pl.ds(off + chunk, chunk)], idx_db, semd_b)
        wait_a()
        pltpu.sync_copy(rows_a, acc.at[idx_da], add=True)

        @pl.when(i + 1 < nb)
        def _():
            pltpu.async_copy(
                g_hbm.at[idx_s.at[pl.ds((2 * i + 2) * chunk, chunk)]],
                rows_a, sem_a)
            pltpu.async_copy(
                dst_hbm.at[pl.ds(off + 2 * chunk, chunk)], idx_da, semd_a)

        g_odd.wait()
        d_odd.wait()
        pltpu.sync_copy(rows_b, acc.at[idx_db], add=True)
        return 0

    lax.fori_loop(0, nb, body, 0)

    if n_extra:
        @pl.when(wid < n_extra)
        def _():
            off = (N_TILES * n_full + wid) * chunk
            pltpu.sync_copy(src_hbm.at[pl.ds(off, chunk)], idx_st)
            pltpu.sync_copy(dst_hbm.at[pl.ds(off, chunk)], idx_dt)
            pltpu.async_copy(g_hbm.at[idx_st], rows_a, sem_a).wait()
            pltpu.sync_copy(rows_a, acc.at[idx_dt], add=True)

    plsc.subcore_barrier()

    pltpu.sync_copy(acc.at[pl.ds(row0, stripe)],
                    part_hbm.at[cid, pl.ds(row0, stripe), :])
    if leftover:
        @pl.when(sid == N_SUBCORES - 1)
        def _():
            pltpu.sync_copy(
                acc.at[pl.ds(stripe * N_SUBCORES, leftover)],
                part_hbm.at[cid, pl.ds(stripe * N_SUBCORES, leftover), :])


def _aggregate(g, src, dst):
    n, ch = g.shape
    E = src.shape[0]
    chunk = 128
    n_full = (E // chunk) // N_TILES
    kern = pl.kernel(
        _agg_body,
        out_type=jax.ShapeDtypeStruct((N_CORES, n, ch), jnp.float32),
        mesh=_MESH,
        scratch_types=[
            pltpu.VMEM((chunk, ch), jnp.float32),
            pltpu.VMEM((chunk, ch), jnp.float32),
            pltpu.VMEM((n_full * chunk,), jnp.int32),
            pltpu.VMEM((chunk,), jnp.int32),
            pltpu.VMEM((chunk,), jnp.int32),
            pltpu.VMEM((chunk,), jnp.int32),
            pltpu.VMEM((chunk,), jnp.int32),
            pltpu.VMEM_SHARED((n, ch), jnp.float32),
            pltpu.SemaphoreType.DMA,
            pltpu.SemaphoreType.DMA,
            pltpu.SemaphoreType.DMA,
            pltpu.SemaphoreType.DMA,
            pltpu.SemaphoreType.DMA,
        ],
    )
    return kern(g, src, dst)


_BLK = 1000


def _first_body(degp_ref, x_ref, w_ref, g_ref, dinv_ref):
    deg = jnp.sum(degp_ref[...], axis=1) + 1.0
    dinv = lax.rsqrt(deg)
    h = jnp.dot(x_ref[...], w_ref[...], preferred_element_type=jnp.float32)
    g_ref[...] = h * dinv[:, None]
    dinv_ref[...] = dinv[:, None]


def _first_stage(degp, x, w):
    n, ch_in = x.shape
    ch_out = w.shape[1]
    grid = (n // _BLK,)
    return pl.pallas_call(
        _first_body,
        grid=grid,
        in_specs=[
            pl.BlockSpec((_BLK, N_TILES), lambda i: (i, 0)),
            pl.BlockSpec((_BLK, ch_in), lambda i: (i, 0)),
            pl.BlockSpec((ch_in, ch_out), lambda i: (0, 0)),
        ],
        out_specs=[
            pl.BlockSpec((_BLK, ch_out), lambda i: (i, 0)),
            pl.BlockSpec((_BLK, 1), lambda i: (i, 0)),
        ],
        out_shape=[
            jax.ShapeDtypeStruct((n, ch_out), jnp.float32),
            jax.ShapeDtypeStruct((n, 1), jnp.float32),
        ],
    )(degp, x, w)


def _mid_body(part_ref, g_ref, dinv_ref, b_ref, w_ref, gn_ref):
    agg = part_ref[0] + part_ref[1] + g_ref[...]
    h = agg * dinv_ref[...] + b_ref[...]
    h = jnp.maximum(h, 0.0)
    hw = jnp.dot(h, w_ref[...], preferred_element_type=jnp.float32)
    gn_ref[...] = hw * dinv_ref[...]


def _mid_stage(part, g, dinv, b, w):
    n, ch = g.shape
    ch_out = w.shape[1]
    grid = (n // _BLK,)
    return pl.pallas_call(
        _mid_body,
        grid=grid,
        in_specs=[
            pl.BlockSpec((N_CORES, _BLK, ch), lambda i: (0, i, 0)),
            pl.BlockSpec((_BLK, ch), lambda i: (i, 0)),
            pl.BlockSpec((_BLK, 1), lambda i: (i, 0)),
            pl.BlockSpec((1, ch), lambda i: (0, 0)),
            pl.BlockSpec((ch, ch_out), lambda i: (0, 0)),
        ],
        out_specs=pl.BlockSpec((_BLK, ch_out), lambda i: (i, 0)),
        out_shape=jax.ShapeDtypeStruct((n, ch_out), jnp.float32),
    )(part, g, dinv, b.reshape(1, ch), w)


def _last_body(part_ref, g_ref, dinv_ref, b_ref, out_ref):
    agg = part_ref[0] + part_ref[1] + g_ref[...]
    out_ref[...] = agg * dinv_ref[...] + b_ref[...]


def _last_stage(part, g, dinv, b):
    n, ch = g.shape
    grid = (n // _BLK,)
    return pl.pallas_call(
        _last_body,
        grid=grid,
        in_specs=[
            pl.BlockSpec((N_CORES, _BLK, ch), lambda i: (0, i, 0)),
            pl.BlockSpec((_BLK, ch), lambda i: (i, 0)),
            pl.BlockSpec((_BLK, 1), lambda i: (i, 0)),
            pl.BlockSpec((1, ch), lambda i: (0, 0)),
        ],
        out_specs=pl.BlockSpec((_BLK, ch), lambda i: (i, 0)),
        out_shape=jax.ShapeDtypeStruct((n, ch), jnp.float32),
    )(part, g, dinv, b.reshape(1, ch))


@jax.jit
def _run(x, edge_index, W1, b1, W2, b2, W_mu, b_mu, W_logstd, b_logstd):
    n = x.shape[0]
    src = edge_index[0]
    dst = edge_index[1]

    degp = _deg_partials(dst, n)

    g1, dinv = _first_stage(degp.T, x, W1)
    part1 = _aggregate(g1, src, dst)

    g2 = _mid_stage(part1, g1, dinv, b1, W2)
    part2 = _aggregate(g2, src, dst)

    w_cat = jnp.concatenate([W_mu, W_logstd], axis=1)
    b_cat = jnp.concatenate([b_mu, b_logstd], axis=0)
    g3 = _mid_stage(part2, g2, dinv, b2, w_cat)
    part3 = _aggregate(g3, src, dst)

    out = _last_stage(part3, g3, dinv, b_cat)
    ch = W_mu.shape[1]
    return out[:, :ch], out[:, ch:]


def kernel(x, edge_index, W1, b1, W2, b2, W_mu, b_mu, W_logstd, b_logstd):
    return _run(x, edge_index, W1, b1, W2, b2, W_mu, b_mu, W_logstd, b_logstd)


# speedup vs baseline: 3.9445x; 1.0242x over previous
import functools

import jax
import jax.numpy as jnp
from jax import lax
from jax.experimental import pallas as pl
from jax.experimental.pallas import tpu as pltpu
from jax.experimental.pallas import tpu_sc as plsc

N_CORES = 2
N_SUBCORES = 16
N_TILES = N_CORES * N_SUBCORES
LANES = 16

_MESH = plsc.VectorSubcoreMesh(core_axis_name="c", subcore_axis_name="s")


def _deg_body(dst_hbm, out_hbm, idx_v, deg_v, sem):
    E = dst_hbm.shape[0]
    n = out_hbm.shape[1]
    per_tile = E // N_TILES
    cid = lax.axis_index("c")
    sid = lax.axis_index("s")
    wid = cid * N_SUBCORES + sid

    pltpu.async_copy(dst_hbm.at[pl.ds(wid * per_tile, per_tile)], idx_v, sem).wait()

    zeros16 = jnp.zeros((LANES,), jnp.float32)
    ones16 = jnp.ones((LANES,), jnp.float32)

    def zero_body(i, _):
        deg_v[pl.ds(i * LANES, LANES)] = zeros16
        return 0

    lax.fori_loop(0, n // LANES, zero_body, 0)

    def acc_body(i, _):
        idx = idx_v[pl.ds(i * LANES, LANES)]
        plsc.addupdate_scatter(deg_v, [idx], ones16)
        return 0

    lax.fori_loop(0, per_tile // LANES, acc_body, 0)

    pltpu.sync_copy(deg_v, out_hbm.at[wid])


def _deg_partials(dst, n_nodes):
    E = dst.shape[0]
    kern = pl.kernel(
        _deg_body,
        out_type=jax.ShapeDtypeStruct((N_TILES, n_nodes), jnp.float32),
        mesh=_MESH,
        compiler_params=pltpu.CompilerParams(needs_layout_passes=False),
        scratch_types=[
            pltpu.VMEM((E // N_TILES,), jnp.int32),
            pltpu.VMEM((n_nodes,), jnp.float32),
            pltpu.SemaphoreType.DMA,
        ],
    )
    return kern(dst)


def _agg_body(g_hbm, src_hbm, dst_hbm, part_hbm,
              rows_a, rows_b, idx_s, idx_da, idx_db,
              idx_st, idx_dt, acc,
              sem_i, sem_a, sem_b, semd_a, semd_b):
    E = src_hbm.shape[0]
    n, ch = g_hbm.shape
    chunk = 128
    n_chunks = E // chunk
    n_full = n_chunks // N_TILES
    n_extra = n_chunks % N_TILES
    stripe = (n // N_SUBCORES) // 8 * 8
    leftover = n - stripe * N_SUBCORES

    cid = lax.axis_index("c")
    sid = lax.axis_index("s")
    wid = cid * N_SUBCORES + sid
    base = wid * n_full * chunk

    i_pend = pltpu.async_copy(
        src_hbm.at[pl.ds(base, n_full * chunk)], idx_s, sem_i)

    zeros16 = jnp.zeros((LANES,), jnp.float32)

    def zrow(i, _):
        for j in range(ch // LANES):
            rows_a[i, pl.ds(j * LANES, LANES)] = zeros16
        return 0

    lax.fori_loop(0, chunk, zrow, 0)

    row0 = sid * stripe
    n_zfull = stripe // chunk
    zrem = stripe % chunk
    for k in range(n_zfull):
        pltpu.sync_copy(rows_a, acc.at[pl.ds(row0 + k * chunk, chunk)])
    if zrem:
        pltpu.sync_copy(rows_a.at[pl.ds(0, zrem)],
                        acc.at[pl.ds(row0 + n_zfull * chunk, zrem)])
    if leftover:
        @pl.when(sid == N_SUBCORES - 1)
        def _():
            pltpu.sync_copy(rows_a.at[pl.ds(0, leftover)],
                            acc.at[pl.ds(stripe * N_SUBCORES, leftover)])

    i_pend.wait()
    pltpu.async_copy(g_hbm.at[idx_s.at[pl.ds(0, chunk)]], rows_a, sem_a)
    pltpu.async_copy(dst_hbm.at[pl.ds(base, chunk)], idx_da, semd_a)

    plsc.subcore_barrier()

    nb = n_full // 2
    assert nb * 2 == n_full

    def wait_a():
        pltpu.make_async_copy(
            g_hbm.at[idx_s.at[pl.ds(0, chunk)]], rows_a, sem_a).wait()
        pltpu.make_async_copy(
            dst_hbm.at[pl.ds(0, chunk)], idx_da, semd_a).wait()

    def body(i, _):
        off = base + 2 * i * chunk
        g_odd = pltpu.async_copy(
            g_hbm.at[idx_s.at[pl.ds((2 * i + 1) * chunk, chunk)]],
            rows_b, sem_b)
        d_odd = pltpu.async_copy(
            dst_hbm.at[pl.ds(off + chunk, chunk)], idx_db, semd_b)
        wait_a()
        pltpu.sync_copy(rows_a, acc.at[idx_da], add=True)

        @pl.when(i + 1 < nb)
        def _():
            pltpu.async_copy(
                g_hbm.at[idx_s.at[pl.ds((2 * i + 2) * chunk, chunk)]],
                rows_a, sem_a)
            pltpu.async_copy(
                dst_hbm.at[pl.ds(off + 2 * chunk, chunk)], idx_da, semd_a)

        g_odd.wait()
        d_odd.wait()
        pltpu.sync_copy(rows_b, acc.at[idx_db], add=True)
        return 0

    lax.fori_loop(0, nb, body, 0)

    if n_extra:
        @pl.when(wid < n_extra)
        def _():
            off = (N_TILES * n_full + wid) * chunk
            pltpu.sync_copy(src_hbm.at[pl.ds(off, chunk)], idx_st)
            pltpu.sync_copy(dst_hbm.at[pl.ds(off, chunk)], idx_dt)
            pltpu.async_copy(g_hbm.at[idx_st], rows_a, sem_a).wait()
            pltpu.sync_copy(rows_a, acc.at[idx_dt], add=True)

    plsc.subcore_barrier()

    pltpu.sync_copy(acc.at[pl.ds(row0, stripe)],
                    part_hbm.at[cid, pl.ds(row0, stripe), :])
    if leftover:
        @pl.when(sid == N_SUBCORES - 1)
        def _():
            pltpu.sync_copy(
                acc.at[pl.ds(stripe * N_SUBCORES, leftover)],
                part_hbm.at[cid, pl.ds(stripe * N_SUBCORES, leftover), :])


def _aggregate(g, src, dst):
    n, ch = g.shape
    E = src.shape[0]
    chunk = 128
    n_full = (E // chunk) // N_TILES
    kern = pl.kernel(
        _agg_body,
        out_type=jax.ShapeDtypeStruct((N_CORES, n, ch), jnp.float32),
        mesh=_MESH,
        scratch_types=[
            pltpu.VMEM((chunk, ch), jnp.float32),
            pltpu.VMEM((chunk, ch), jnp.float32),
            pltpu.VMEM((n_full * chunk,), jnp.int32),
            pltpu.VMEM((chunk,), jnp.int32),
            pltpu.VMEM((chunk,), jnp.int32),
            pltpu.VMEM((chunk,), jnp.int32),
            pltpu.VMEM((chunk,), jnp.int32),
            pltpu.VMEM_SHARED((n, ch), jnp.float32),
            pltpu.SemaphoreType.DMA,
            pltpu.SemaphoreType.DMA,
            pltpu.SemaphoreType.DMA,
            pltpu.SemaphoreType.DMA,
            pltpu.SemaphoreType.DMA,
        ],
    )
    return kern(g, src, dst)


_BLK = 2000


def _first_body(degp_ref, x_ref, w_ref, g_ref, dinv_ref):
    deg = jnp.sum(degp_ref[...], axis=1) + 1.0
    dinv = lax.rsqrt(deg)
    h = jnp.dot(x_ref[...], w_ref[...], preferred_element_type=jnp.float32)
    g_ref[...] = h * dinv[:, None]
    dinv_ref[...] = dinv[:, None]


def _first_stage(degp, x, w):
    n, ch_in = x.shape
    ch_out = w.shape[1]
    grid = (n // _BLK,)
    return pl.pallas_call(
        _first_body,
        grid=grid,
        in_specs=[
            pl.BlockSpec((_BLK, N_TILES), lambda i: (i, 0)),
            pl.BlockSpec((_BLK, ch_in), lambda i: (i, 0)),
            pl.BlockSpec((ch_in, ch_out), lambda i: (0, 0)),
        ],
        out_specs=[
            pl.BlockSpec((_BLK, ch_out), lambda i: (i, 0)),
            pl.BlockSpec((_BLK, 1), lambda i: (i, 0)),
        ],
        out_shape=[
            jax.ShapeDtypeStruct((n, ch_out), jnp.float32),
            jax.ShapeDtypeStruct((n, 1), jnp.float32),
        ],
    )(degp, x, w)


def _mid_body(part_ref, g_ref, dinv_ref, b_ref, w_ref, gn_ref):
    agg = part_ref[0] + part_ref[1] + g_ref[...]
    h = agg * dinv_ref[...] + b_ref[...]
    h = jnp.maximum(h, 0.0)
    hw = jnp.dot(h, w_ref[...], preferred_element_type=jnp.float32)
    gn_ref[...] = hw * dinv_ref[...]


def _mid_stage(part, g, dinv, b, w):
    n, ch = g.shape
    ch_out = w.shape[1]
    grid = (n // _BLK,)
    return pl.pallas_call(
        _mid_body,
        grid=grid,
        in_specs=[
            pl.BlockSpec((N_CORES, _BLK, ch), lambda i: (0, i, 0)),
            pl.BlockSpec((_BLK, ch), lambda i: (i, 0)),
            pl.BlockSpec((_BLK, 1), lambda i: (i, 0)),
            pl.BlockSpec((1, ch), lambda i: (0, 0)),
            pl.BlockSpec((ch, ch_out), lambda i: (0, 0)),
        ],
        out_specs=pl.BlockSpec((_BLK, ch_out), lambda i: (i, 0)),
        out_shape=jax.ShapeDtypeStruct((n, ch_out), jnp.float32),
    )(part, g, dinv, b.reshape(1, ch), w)


def _last_body(part_ref, g_ref, dinv_ref, b_ref, out_ref):
    agg = part_ref[0] + part_ref[1] + g_ref[...]
    out_ref[...] = agg * dinv_ref[...] + b_ref[...]


def _last_stage(part, g, dinv, b):
    n, ch = g.shape
    grid = (n // _BLK,)
    return pl.pallas_call(
        _last_body,
        grid=grid,
        in_specs=[
            pl.BlockSpec((N_CORES, _BLK, ch), lambda i: (0, i, 0)),
            pl.BlockSpec((_BLK, ch), lambda i: (i, 0)),
            pl.BlockSpec((_BLK, 1), lambda i: (i, 0)),
            pl.BlockSpec((1, ch), lambda i: (0, 0)),
        ],
        out_specs=pl.BlockSpec((_BLK, ch), lambda i: (i, 0)),
        out_shape=jax.ShapeDtypeStruct((n, ch), jnp.float32),
    )(part, g, dinv, b.reshape(1, ch))


@jax.jit
def _run(x, edge_index, W1, b1, W2, b2, W_mu, b_mu, W_logstd, b_logstd):
    n = x.shape[0]
    src = edge_index[0]
    dst = edge_index[1]

    degp = _deg_partials(dst, n)

    g1, dinv = _first_stage(degp.T, x, W1)
    part1 = _aggregate(g1, src, dst)

    g2 = _mid_stage(part1, g1, dinv, b1, W2)
    part2 = _aggregate(g2, src, dst)

    w_cat = jnp.concatenate([W_mu, W_logstd], axis=1)
    b_cat = jnp.concatenate([b_mu, b_logstd], axis=0)
    g3 = _mid_stage(part2, g2, dinv, b2, w_cat)
    part3 = _aggregate(g3, src, dst)

    out = _last_stage(part3, g3, dinv, b_cat)
    ch = W_mu.shape[1]
    return out[:, :ch], out[:, ch:]


def kernel(x, edge_index, W1, b1, W2, b2, W_mu, b_mu, W_logstd, b_logstd):
    return _run(x, edge_index, W1, b1, W2, b2, W_mu, b_mu, W_logstd, b_logstd)
